# initial kernel scaffold (unmeasured)
import functools

import jax
import jax.numpy as jnp
from jax import lax
from jax.experimental import pallas as pl
from jax.experimental.pallas import tpu as pltpu

B, S, H, Dh, Dr = 4, 256, 32, 128, 64
M = B * S
D = 4096
DC = 128
BF = jnp.bfloat16
F32 = jnp.float32
SCALE = (Dh + Dr) ** -0.5


def _c_kr_body(x_ref, wdkv_ref, wkr_ref, c_ref, kr_ref):
    xv = x_ref[...]
    c_ref[...] = jnp.dot(
        xv, wdkv_ref[...].astype(BF), preferred_element_type=F32
    ).astype(BF)
    kr_ref[...] = jnp.dot(
        xv, wkr_ref[...].astype(BF), preferred_element_type=F32
    ).astype(BF)


def _c_kr(xb, wdkv, wkr):
    return pl.pallas_call(
        _c_kr_body,
        out_shape=[
            jax.ShapeDtypeStruct((M, DC), BF),
            jax.ShapeDtypeStruct((M, Dr), BF),
        ],
        in_specs=[
            pl.BlockSpec(memory_space=pltpu.VMEM),
            pl.BlockSpec(memory_space=pltpu.VMEM),
            pl.BlockSpec(memory_space=pltpu.VMEM),
        ],
        out_specs=[
            pl.BlockSpec(memory_space=pltpu.VMEM),
            pl.BlockSpec(memory_space=pltpu.VMEM),
        ],
    )(xb, wdkv, wkr)


def _exchange_kv_body(
    c_ref, wuk_ref, wuv_ref, k_ref, v_ref,
    c_full, wuk_full, wuv_full, send_sems, recv_sems,
):
    my_x = lax.axis_index("x")
    my_y = lax.axis_index("y")
    my_z = lax.axis_index("z")
    partner = (1 - my_x, my_y, my_z)

    c_full[my_x] = c_ref[...]
    wuk_full[my_x] = wuk_ref[...].astype(BF)
    wuv_full[my_x] = wuv_ref[...].astype(BF)

    barrier = pltpu.get_barrier_semaphore()
    pl.semaphore_signal(
        barrier, inc=1, device_id=partner, device_id_type=pl.DeviceIdType.MESH
    )
    pl.semaphore_wait(barrier, 1)

    rdmas = []
    for i, buf in enumerate((c_full, wuk_full, wuv_full)):
        rdma = pltpu.make_async_remote_copy(
            src_ref=buf.at[my_x],
            dst_ref=buf.at[my_x],
            send_sem=send_sems.at[i],
            recv_sem=recv_sems.at[i],
            device_id=partner,
            device_id_type=pl.DeviceIdType.MESH,
        )
        rdma.start()
        rdmas.append(rdma)
    for rdma in rdmas:
        rdma.wait()

    k_ref[...] = (
        jnp.dot(c_full[0], wuk_full[0], preferred_element_type=F32)
        + jnp.dot(c_full[1], wuk_full[1], preferred_element_type=F32)
    ).astype(BF)
    v_ref[...] = (
        jnp.dot(c_full[0], wuv_full[0], preferred_element_type=F32)
        + jnp.dot(c_full[1], wuv_full[1], preferred_element_type=F32)
    ).astype(BF)


def _exchange_kv(c, wuk, wuv):
    return pl.pallas_call(
        _exchange_kv_body,
        out_shape=[
            jax.ShapeDtypeStruct((M, D), BF),
            jax.ShapeDtypeStruct((M, D), BF),
        ],
        in_specs=[
            pl.BlockSpec(memory_space=pltpu.VMEM),
            pl.BlockSpec(memory_space=pltpu.VMEM),
            pl.BlockSpec(memory_space=pltpu.VMEM),
        ],
        out_specs=[
            pl.BlockSpec(memory_space=pltpu.VMEM),
            pl.BlockSpec(memory_space=pltpu.VMEM),
        ],
        scratch_shapes=[
            pltpu.VMEM((2, M, DC), BF),
            pltpu.VMEM((2, DC, D), BF),
            pltpu.VMEM((2, DC, D), BF),
            pltpu.SemaphoreType.DMA((3,)),
            pltpu.SemaphoreType.DMA((3,)),
        ],
        compiler_params=pltpu.CompilerParams(collective_id=0),
    )(c, wuk, wuv)


def _mm_body(a_ref, w_ref, o_ref, *, out_dtype):
    o_ref[...] = jnp.dot(
        a_ref[...], w_ref[...].astype(BF), preferred_element_type=F32
    ).astype(out_dtype)


def _mm(a, w, block_n, out_dtype):
    m, k = a.shape
    _, n = w.shape
    return pl.pallas_call(
        functools.partial(_mm_body, out_dtype=out_dtype),
        grid=(n // block_n,),
        in_specs=[
            pl.BlockSpec((m, k), lambda j: (0, 0)),
            pl.BlockSpec((k, block_n), lambda j: (0, j)),
        ],
        out_specs=pl.BlockSpec((m, block_n), lambda j: (0, j)),
        out_shape=jax.ShapeDtypeStruct((m, n), out_dtype),
    )(a, w)


def _attn_body(q_ref, k_ref, v_ref, qr_ref, kr_ref, o_ref):
    q = q_ref[...]
    k = k_ref[...]
    v = v_ref[...]
    s = lax.dot_general(
        q, k, (((1,), (1,)), ((), ())), preferred_element_type=F32
    )
    s = s + lax.dot_general(
        qr_ref[...], kr_ref[...], (((1,), (1,)), ((), ())),
        preferred_element_type=F32,
    )
    s = s * SCALE
    mx = jnp.max(s, axis=1, keepdims=True)
    p = jnp.exp(s - mx)
    p = p / jnp.sum(p, axis=1, keepdims=True)
    o_ref[...] = jnp.dot(
        p.astype(BF), v, preferred_element_type=F32
    ).astype(BF)


def _attention(q, k, v, qr, kr):
    return pl.pallas_call(
        _attn_body,
        grid=(B, H),
        in_specs=[
            pl.BlockSpec((S, Dh), lambda b, h: (b, h)),
            pl.BlockSpec((S, Dh), lambda b, h: (b, h)),
            pl.BlockSpec((S, Dh), lambda b, h: (b, h)),
            pl.BlockSpec((S, Dr), lambda b, h: (b, h)),
            pl.BlockSpec((S, Dr), lambda b, h: (b, 0)),
        ],
        out_specs=pl.BlockSpec((S, Dh), lambda b, h: (b, h)),
        out_shape=jax.ShapeDtypeStruct((M, H * Dh), BF),
    )(q, k, v, qr, kr)


def kernel(x, Wdkv, Wuk, Wuv, Wq, Wqr, Wkr, Wo):
    xb = x.reshape(M, D).astype(BF)
    c, kr = _c_kr(xb, Wdkv, Wkr)
    k, v = _exchange_kv(c, Wuk, Wuv)
    q = _mm(xb, Wq, 512, BF)
    qr = _mm(xb, Wqr, 512, BF)
    o = _attention(q, k, v, qr, kr)
    out = _mm(o, Wo, 512, F32)
    return out.reshape(B, S, D)


# baseline (device time: 285342 ns/iter reference)
import functools

import jax
import jax.numpy as jnp
from jax import lax
from jax.experimental import pallas as pl
from jax.experimental.pallas import tpu as pltpu

B, S, H, Dh, Dr = 4, 256, 32, 128, 64
M = B * S
D = 4096
DC = 128
BF = jnp.bfloat16
F32 = jnp.float32
SCALE = (Dh + Dr) ** -0.5


def _c_kr_body(x_ref, wdkv_ref, wkr_ref, c_ref, kr_ref):
    xv = x_ref[...]
    c_ref[...] = jnp.dot(
        xv, wdkv_ref[...].astype(BF), preferred_element_type=F32
    ).astype(BF)
    kr_ref[...] = jnp.dot(
        xv, wkr_ref[...].astype(BF), preferred_element_type=F32
    ).astype(BF)


def _c_kr(xb, wdkv, wkr):
    return pl.pallas_call(
        _c_kr_body,
        out_shape=[
            jax.ShapeDtypeStruct((M, DC), BF),
            jax.ShapeDtypeStruct((M, Dr), BF),
        ],
        in_specs=[
            pl.BlockSpec(memory_space=pltpu.VMEM),
            pl.BlockSpec(memory_space=pltpu.VMEM),
            pl.BlockSpec(memory_space=pltpu.VMEM),
        ],
        out_specs=[
            pl.BlockSpec(memory_space=pltpu.VMEM),
            pl.BlockSpec(memory_space=pltpu.VMEM),
        ],
    )(xb, wdkv, wkr)


def _exchange_kv_body(
    c_ref, wuk_ref, wuv_ref, k_ref, v_ref,
    c_full, wuk_full, wuv_full, send_sems, recv_sems,
):
    my_x = lax.axis_index("x")
    my_y = lax.axis_index("y")
    my_z = lax.axis_index("z")
    partner = (1 - my_x, my_y, my_z)

    c_full[my_x] = c_ref[...]
    wuk_full[my_x] = wuk_ref[...].astype(BF)
    wuv_full[my_x] = wuv_ref[...].astype(BF)

    barrier = pltpu.get_barrier_semaphore()
    pl.semaphore_signal(
        barrier, inc=1, device_id=partner, device_id_type=pl.DeviceIdType.MESH
    )
    pl.semaphore_wait(barrier, 1)

    rdmas = []
    for i, buf in enumerate((c_full, wuk_full, wuv_full)):
        rdma = pltpu.make_async_remote_copy(
            src_ref=buf.at[my_x],
            dst_ref=buf.at[my_x],
            send_sem=send_sems.at[i],
            recv_sem=recv_sems.at[i],
            device_id=partner,
            device_id_type=pl.DeviceIdType.MESH,
        )
        rdma.start()
        rdmas.append(rdma)
    for rdma in rdmas:
        rdma.wait()

    k_ref[...] = (
        jnp.dot(c_full[0], wuk_full[0], preferred_element_type=F32)
        + jnp.dot(c_full[1], wuk_full[1], preferred_element_type=F32)
    ).astype(BF)
    v_ref[...] = (
        jnp.dot(c_full[0], wuv_full[0], preferred_element_type=F32)
        + jnp.dot(c_full[1], wuv_full[1], preferred_element_type=F32)
    ).astype(BF)


def _exchange_kv(c, wuk, wuv):
    return pl.pallas_call(
        _exchange_kv_body,
        out_shape=[
            jax.ShapeDtypeStruct((M, D), BF),
            jax.ShapeDtypeStruct((M, D), BF),
        ],
        in_specs=[
            pl.BlockSpec(memory_space=pltpu.VMEM),
            pl.BlockSpec(memory_space=pltpu.VMEM),
            pl.BlockSpec(memory_space=pltpu.VMEM),
        ],
        out_specs=[
            pl.BlockSpec(memory_space=pltpu.VMEM),
            pl.BlockSpec(memory_space=pltpu.VMEM),
        ],
        scratch_shapes=[
            pltpu.VMEM((2, M, DC), BF),
            pltpu.VMEM((2, DC, D), BF),
            pltpu.VMEM((2, DC, D), BF),
            pltpu.SemaphoreType.DMA((3,)),
            pltpu.SemaphoreType.DMA((3,)),
        ],
        compiler_params=pltpu.CompilerParams(collective_id=0),
    )(c, wuk, wuv)


def _mm_body(a_ref, w_ref, o_ref, *, out_dtype):
    o_ref[...] = jnp.dot(
        a_ref[...], w_ref[...].astype(BF), preferred_element_type=F32
    ).astype(out_dtype)


def _mm(a, w, block_n, out_dtype):
    m, k = a.shape
    _, n = w.shape
    return pl.pallas_call(
        functools.partial(_mm_body, out_dtype=out_dtype),
        grid=(n // block_n,),
        in_specs=[
            pl.BlockSpec((m, k), lambda j: (0, 0)),
            pl.BlockSpec((k, block_n), lambda j: (0, j)),
        ],
        out_specs=pl.BlockSpec((m, block_n), lambda j: (0, j)),
        out_shape=jax.ShapeDtypeStruct((m, n), out_dtype),
    )(a, w)


def _attn_body(q_ref, k_ref, v_ref, qr_ref, kr_ref, o_ref):
    q = q_ref[...]
    k = k_ref[...]
    v = v_ref[...]
    s = lax.dot_general(
        q, k, (((1,), (1,)), ((), ())), preferred_element_type=F32
    )
    s = s + lax.dot_general(
        qr_ref[0], kr_ref[...], (((1,), (1,)), ((), ())),
        preferred_element_type=F32,
    )
    s = s * SCALE
    mx = jnp.max(s, axis=1, keepdims=True)
    p = jnp.exp(s - mx)
    p = p / jnp.sum(p, axis=1, keepdims=True)
    o_ref[...] = jnp.dot(
        p.astype(BF), v, preferred_element_type=F32
    ).astype(BF)


def _attention(q, k, v, qr, kr):
    return pl.pallas_call(
        _attn_body,
        grid=(B, H),
        in_specs=[
            pl.BlockSpec((S, Dh), lambda b, h: (b, h)),
            pl.BlockSpec((S, Dh), lambda b, h: (b, h)),
            pl.BlockSpec((S, Dh), lambda b, h: (b, h)),
            pl.BlockSpec((1, S, Dr), lambda b, h: (h, b, 0)),
            pl.BlockSpec((S, Dr), lambda b, h: (b, 0)),
        ],
        out_specs=pl.BlockSpec((S, Dh), lambda b, h: (b, h)),
        out_shape=jax.ShapeDtypeStruct((M, H * Dh), BF),
    )(q, k, v, qr, kr)


def kernel(x, Wdkv, Wuk, Wuv, Wq, Wqr, Wkr, Wo):
    xb = x.reshape(M, D).astype(BF)
    c, kr = _c_kr(xb, Wdkv, Wkr)
    k, v = _exchange_kv(c, Wuk, Wuv)
    q = _mm(xb, Wq, 512, BF)
    qr = _mm(xb, Wqr, 512, BF)
    qr3 = qr.reshape(M, H, Dr).transpose(1, 0, 2)
    o = _attention(q, k, v, qr3, kr)
    out = _mm(o, Wo, 512, F32)
    return out.reshape(B, S, D)


# device time: 237115 ns/iter; 1.2034x vs baseline; 1.2034x over previous
import functools

import jax
import jax.numpy as jnp
from jax import lax
from jax.experimental import pallas as pl
from jax.experimental.pallas import tpu as pltpu

B, S, H, Dh, Dr = 4, 256, 32, 128, 64
M = B * S
D = 4096
DC = 128
BF = jnp.bfloat16
F32 = jnp.float32
SCALE = (Dh + Dr) ** -0.5


def _c_kr_body(x_ref, wdkv_ref, wkr_ref, c_ref, kr_ref):
    xv = x_ref[...]
    c_ref[...] = jnp.dot(
        xv, wdkv_ref[...].astype(BF), preferred_element_type=F32
    ).astype(BF)
    kr_ref[...] = jnp.dot(
        xv, wkr_ref[...].astype(BF), preferred_element_type=F32
    ).astype(BF)


def _c_kr(xb, wdkv, wkr):
    return pl.pallas_call(
        _c_kr_body,
        out_shape=[
            jax.ShapeDtypeStruct((M, DC), BF),
            jax.ShapeDtypeStruct((M, Dr), BF),
        ],
        in_specs=[
            pl.BlockSpec(memory_space=pltpu.VMEM),
            pl.BlockSpec(memory_space=pltpu.VMEM),
            pl.BlockSpec(memory_space=pltpu.VMEM),
        ],
        out_specs=[
            pl.BlockSpec(memory_space=pltpu.VMEM),
            pl.BlockSpec(memory_space=pltpu.VMEM),
        ],
    )(xb, wdkv, wkr)


def _exchange_kv_body(
    c_ref, wuk_ref, wuv_ref, k_ref, v_ref,
    c_full, wuk_full, wuv_full, send_sems, recv_sems,
):
    my_x = lax.axis_index("x")
    my_y = lax.axis_index("y")
    my_z = lax.axis_index("z")
    partner = (1 - my_x, my_y, my_z)

    c_full[my_x] = c_ref[...]
    wuk_full[my_x] = wuk_ref[...].astype(BF)
    wuv_full[my_x] = wuv_ref[...].astype(BF)

    barrier = pltpu.get_barrier_semaphore()
    pl.semaphore_signal(
        barrier, inc=1, device_id=partner, device_id_type=pl.DeviceIdType.MESH
    )
    pl.semaphore_wait(barrier, 1)

    rdmas = []
    for i, buf in enumerate((c_full, wuk_full, wuv_full)):
        rdma = pltpu.make_async_remote_copy(
            src_ref=buf.at[my_x],
            dst_ref=buf.at[my_x],
            send_sem=send_sems.at[i],
            recv_sem=recv_sems.at[i],
            device_id=partner,
            device_id_type=pl.DeviceIdType.MESH,
        )
        rdma.start()
        rdmas.append(rdma)
    for rdma in rdmas:
        rdma.wait()

    k_ref[...] = (
        jnp.dot(c_full[0], wuk_full[0], preferred_element_type=F32)
        + jnp.dot(c_full[1], wuk_full[1], preferred_element_type=F32)
    ).astype(BF)
    v_ref[...] = (
        jnp.dot(c_full[0], wuv_full[0], preferred_element_type=F32)
        + jnp.dot(c_full[1], wuv_full[1], preferred_element_type=F32)
    ).astype(BF)


def _exchange_kv(c, wuk, wuv):
    return pl.pallas_call(
        _exchange_kv_body,
        out_shape=[
            jax.ShapeDtypeStruct((M, D), BF),
            jax.ShapeDtypeStruct((M, D), BF),
        ],
        in_specs=[
            pl.BlockSpec(memory_space=pltpu.VMEM),
            pl.BlockSpec(memory_space=pltpu.VMEM),
            pl.BlockSpec(memory_space=pltpu.VMEM),
        ],
        out_specs=[
            pl.BlockSpec(memory_space=pltpu.VMEM),
            pl.BlockSpec(memory_space=pltpu.VMEM),
        ],
        scratch_shapes=[
            pltpu.VMEM((2, M, DC), BF),
            pltpu.VMEM((2, DC, D), BF),
            pltpu.VMEM((2, DC, D), BF),
            pltpu.SemaphoreType.DMA((3,)),
            pltpu.SemaphoreType.DMA((3,)),
        ],
        compiler_params=pltpu.CompilerParams(collective_id=0),
    )(c, wuk, wuv)


def _mm_body(a_ref, w_ref, o_ref, *, out_dtype):
    o_ref[...] = jnp.dot(
        a_ref[...], w_ref[...].astype(BF), preferred_element_type=F32
    ).astype(out_dtype)


def _mm(a, w, block_n, out_dtype):
    m, k = a.shape
    _, n = w.shape
    return pl.pallas_call(
        functools.partial(_mm_body, out_dtype=out_dtype),
        grid=(n // block_n,),
        in_specs=[
            pl.BlockSpec((m, k), lambda j: (0, 0)),
            pl.BlockSpec((k, block_n), lambda j: (0, j)),
        ],
        out_specs=pl.BlockSpec((m, block_n), lambda j: (0, j)),
        out_shape=jax.ShapeDtypeStruct((m, n), out_dtype),
    )(a, w)


def _attn_body(q_ref, k_ref, v_ref, qr_ref, kr_ref, o_ref):
    qr = qr_ref[...]
    kr = kr_ref[...]
    for h in range(H):
        q = q_ref[:, h * Dh:(h + 1) * Dh]
        k = k_ref[:, h * Dh:(h + 1) * Dh]
        s = lax.dot_general(
            q, k, (((1,), (1,)), ((), ())), preferred_element_type=F32
        )
        s = s + lax.dot_general(
            qr[:, h * Dr:(h + 1) * Dr], kr, (((1,), (1,)), ((), ())),
            preferred_element_type=F32,
        )
        s = s * SCALE
        mx = jnp.max(s, axis=1, keepdims=True)
        p = jnp.exp(s - mx)
        p = p / jnp.sum(p, axis=1, keepdims=True)
        o_ref[:, h * Dh:(h + 1) * Dh] = jnp.dot(
            p.astype(BF), v_ref[:, h * Dh:(h + 1) * Dh],
            preferred_element_type=F32,
        ).astype(BF)


def _attention(q, k, v, qr, kr):
    return pl.pallas_call(
        _attn_body,
        grid=(B,),
        in_specs=[
            pl.BlockSpec((S, H * Dh), lambda b: (b, 0)),
            pl.BlockSpec((S, H * Dh), lambda b: (b, 0)),
            pl.BlockSpec((S, H * Dh), lambda b: (b, 0)),
            pl.BlockSpec((S, H * Dr), lambda b: (b, 0)),
            pl.BlockSpec((S, Dr), lambda b: (b, 0)),
        ],
        out_specs=pl.BlockSpec((S, H * Dh), lambda b: (b, 0)),
        out_shape=jax.ShapeDtypeStruct((M, H * Dh), BF),
    )(q, k, v, qr, kr)


def kernel(x, Wdkv, Wuk, Wuv, Wq, Wqr, Wkr, Wo):
    xb = x.reshape(M, D).astype(BF)
    c, kr = _c_kr(xb, Wdkv, Wkr)
    k, v = _exchange_kv(c, Wuk, Wuv)
    q = _mm(xb, Wq, 512, BF)
    qr = _mm(xb, Wqr, 512, BF)
    o = _attention(q, k, v, qr, kr)
    out = _mm(o, Wo, 512, F32)
    return out.reshape(B, S, D)


# device time: 182029 ns/iter; 1.5676x vs baseline; 1.3026x over previous
import functools

import jax
import jax.numpy as jnp
from jax import lax
from jax.experimental import pallas as pl
from jax.experimental.pallas import tpu as pltpu

B, S, H, Dh, Dr = 4, 256, 32, 128, 64
M = B * S
D = 4096
DC = 128
BF = jnp.bfloat16
F32 = jnp.float32
SCALE = (Dh + Dr) ** -0.5


def _c_kr_body(x_ref, wdkv_ref, wkr_ref, c_ref, kr_ref):
    xv = x_ref[...]
    c_ref[...] = jnp.dot(
        xv, wdkv_ref[...].astype(BF), preferred_element_type=F32
    ).astype(BF)
    kr_ref[...] = jnp.dot(
        xv, wkr_ref[...].astype(BF), preferred_element_type=F32
    ).astype(BF)


def _c_kr(xb, wdkv, wkr):
    return pl.pallas_call(
        _c_kr_body,
        out_shape=[
            jax.ShapeDtypeStruct((M, DC), BF),
            jax.ShapeDtypeStruct((M, Dr), BF),
        ],
        in_specs=[
            pl.BlockSpec(memory_space=pltpu.VMEM),
            pl.BlockSpec(memory_space=pltpu.VMEM),
            pl.BlockSpec(memory_space=pltpu.VMEM),
        ],
        out_specs=[
            pl.BlockSpec(memory_space=pltpu.VMEM),
            pl.BlockSpec(memory_space=pltpu.VMEM),
        ],
    )(xb, wdkv, wkr)


QR_BN = 256


def _exchange_qr_kv_body(
    c_ref, wuk_ref, wuv_ref, xb_ref, wqr_ref,
    k_ref, v_ref, qr_ref,
    c_full, wuk_full, wuv_full, send_sems, recv_sems,
):
    j = pl.program_id(0)
    nj = pl.num_programs(0)
    my_x = lax.axis_index("x")
    my_y = lax.axis_index("y")
    my_z = lax.axis_index("z")
    partner = (1 - my_x, my_y, my_z)

    def make_rdmas():
        rdmas = []
        for i, buf in enumerate((c_full, wuk_full, wuv_full)):
            rdmas.append(
                pltpu.make_async_remote_copy(
                    src_ref=buf.at[my_x],
                    dst_ref=buf.at[my_x],
                    send_sem=send_sems.at[i],
                    recv_sem=recv_sems.at[i],
                    device_id=partner,
                    device_id_type=pl.DeviceIdType.MESH,
                )
            )
        return rdmas

    @pl.when(j == 0)
    def _():
        c_full[my_x] = c_ref[...]
        wuk_full[my_x] = wuk_ref[...].astype(BF)
        wuv_full[my_x] = wuv_ref[...].astype(BF)
        barrier = pltpu.get_barrier_semaphore()
        pl.semaphore_signal(
            barrier, inc=1, device_id=partner,
            device_id_type=pl.DeviceIdType.MESH,
        )
        pl.semaphore_wait(barrier, 1)
        for rdma in make_rdmas():
            rdma.start()

    qr_ref[...] = (
        jnp.dot(xb_ref[...], wqr_ref[...].astype(BF),
                preferred_element_type=F32) * SCALE
    ).astype(BF)

    @pl.when(j == nj - 1)
    def _():
        for rdma in make_rdmas():
            rdma.wait()
        ck = 1024
        for col in range(0, D, ck):
            cs = slice(col, col + ck)
            k_ref[:, cs] = (
                jnp.dot(c_full[0], wuk_full[0][:, cs],
                        preferred_element_type=F32)
                + jnp.dot(c_full[1], wuk_full[1][:, cs],
                          preferred_element_type=F32)
            ).astype(BF)
            v_ref[:, cs] = (
                jnp.dot(c_full[0], wuv_full[0][:, cs],
                        preferred_element_type=F32)
                + jnp.dot(c_full[1], wuv_full[1][:, cs],
                          preferred_element_type=F32)
            ).astype(BF)


def _exchange_qr_kv(c, wuk, wuv, xb, wqr):
    nqr = H * Dr
    return pl.pallas_call(
        _exchange_qr_kv_body,
        grid=(nqr // QR_BN,),
        in_specs=[
            pl.BlockSpec((M, DC), lambda j: (0, 0)),
            pl.BlockSpec((DC, D), lambda j: (0, 0)),
            pl.BlockSpec((DC, D), lambda j: (0, 0)),
            pl.BlockSpec((M, D), lambda j: (0, 0)),
            pl.BlockSpec((D, QR_BN), lambda j: (0, j)),
        ],
        out_specs=[
            pl.BlockSpec((M, D), lambda j: (0, 0)),
            pl.BlockSpec((M, D), lambda j: (0, 0)),
            pl.BlockSpec((M, QR_BN), lambda j: (0, j)),
        ],
        out_shape=[
            jax.ShapeDtypeStruct((M, D), BF),
            jax.ShapeDtypeStruct((M, D), BF),
            jax.ShapeDtypeStruct((M, nqr), BF),
        ],
        scratch_shapes=[
            pltpu.VMEM((2, M, DC), BF),
            pltpu.VMEM((2, DC, D), BF),
            pltpu.VMEM((2, DC, D), BF),
            pltpu.SemaphoreType.DMA((3,)),
            pltpu.SemaphoreType.DMA((3,)),
        ],
        compiler_params=pltpu.CompilerParams(
            collective_id=0, vmem_limit_bytes=56 * 1024 * 1024
        ),
    )(c, wuk, wuv, xb, wqr)


def _mm_body(a_ref, w_ref, o_ref, *, out_dtype, scale):
    acc = jnp.dot(
        a_ref[...], w_ref[...].astype(BF), preferred_element_type=F32
    )
    if scale is not None:
        acc = acc * scale
    o_ref[...] = acc.astype(out_dtype)


def _mm(a, w, block_n, out_dtype, scale=None):
    m, k = a.shape
    _, n = w.shape
    return pl.pallas_call(
        functools.partial(_mm_body, out_dtype=out_dtype, scale=scale),
        grid=(n // block_n,),
        in_specs=[
            pl.BlockSpec((m, k), lambda j: (0, 0)),
            pl.BlockSpec((k, block_n), lambda j: (0, j)),
        ],
        out_specs=pl.BlockSpec((m, block_n), lambda j: (0, j)),
        out_shape=jax.ShapeDtypeStruct((m, n), out_dtype),
    )(a, w)


def _attn_body(q_ref, k_ref, v_ref, qr_ref, kr_ref, o_ref):
    qr = qr_ref[...]
    kr = kr_ref[...]
    for h in range(H):
        qcat = jnp.concatenate(
            [q_ref[:, h * Dh:(h + 1) * Dh], qr[:, h * Dr:(h + 1) * Dr]],
            axis=1,
        )
        kcat = jnp.concatenate([k_ref[:, h * Dh:(h + 1) * Dh], kr], axis=1)
        s = lax.dot_general(
            qcat, kcat, (((1,), (1,)), ((), ())), preferred_element_type=F32
        )
        mx = jnp.max(s, axis=1, keepdims=True)
        p = jnp.exp(s - mx)
        p = p / jnp.sum(p, axis=1, keepdims=True)
        o_ref[:, h * Dh:(h + 1) * Dh] = jnp.dot(
            p.astype(BF), v_ref[:, h * Dh:(h + 1) * Dh],
            preferred_element_type=F32,
        ).astype(BF)


def _attention(q, k, v, qr, kr):
    return pl.pallas_call(
        _attn_body,
        grid=(B,),
        in_specs=[
            pl.BlockSpec((S, H * Dh), lambda b: (b, 0)),
            pl.BlockSpec((S, H * Dh), lambda b: (b, 0)),
            pl.BlockSpec((S, H * Dh), lambda b: (b, 0)),
            pl.BlockSpec((S, H * Dr), lambda b: (b, 0)),
            pl.BlockSpec((S, Dr), lambda b: (b, 0)),
        ],
        out_specs=pl.BlockSpec((S, H * Dh), lambda b: (b, 0)),
        out_shape=jax.ShapeDtypeStruct((M, H * Dh), BF),
    )(q, k, v, qr, kr)


def kernel(x, Wdkv, Wuk, Wuv, Wq, Wqr, Wkr, Wo):
    xb = x.reshape(M, D).astype(BF)
    c, kr = _c_kr(xb, Wdkv, Wkr)
    k, v, qr = _exchange_qr_kv(c, Wuk, Wuv, xb, Wqr)
    q = _mm(xb, Wq, 512, BF, scale=SCALE)
    o = _attention(q, k, v, qr, kr)
    out = _mm(o, Wo, 512, F32)
    return out.reshape(B, S, D)
